# baseline (speedup 1.0000x reference)
import functools

import jax
import jax.numpy as jnp
from jax.experimental import pallas as pl
from jax.experimental.pallas import tpu as pltpu

_K = 32
_EXPAND2 = 0.01


def _expand_body(qpts_ref, ptsT_ref, frow_ref, fcol_ref, out_ref):
    q = qpts_ref[0]
    pT = ptsT_ref[0]
    frow = frow_ref[0]
    m0row = frow > 0.0
    filtT = jnp.where(m0row, pT, 1e6)
    qn = jnp.sum(q * q, axis=1, keepdims=True)
    fn = jnp.sum(filtT * filtT, axis=0, keepdims=True)
    cross = jnp.dot(q, filtT, preferred_element_type=jnp.float32)
    d2 = qn + fn - 2.0 * cross
    nn = jnp.min(d2, axis=1, keepdims=True)
    m0q = fcol_ref[0] > 0.0
    emask = jnp.logical_or(m0q, nn < _EXPAND2)
    out_ref[0] = emask.astype(jnp.float32)


def _expand_mask(points, features):
    B, N, _ = points.shape
    Nb = 256
    ptsT = jnp.swapaxes(points, 1, 2)
    frow = jnp.swapaxes(features, 1, 2)
    return pl.pallas_call(
        _expand_body,
        grid=(B, N // Nb),
        in_specs=[
            pl.BlockSpec((1, Nb, 3), lambda b, i: (b, i, 0)),
            pl.BlockSpec((1, 3, N), lambda b, i: (b, 0, 0)),
            pl.BlockSpec((1, 1, N), lambda b, i: (b, 0, 0)),
            pl.BlockSpec((1, Nb, 1), lambda b, i: (b, i, 0)),
        ],
        out_specs=pl.BlockSpec((1, Nb, 1), lambda b, i: (b, i, 0)),
        out_shape=jax.ShapeDtypeStruct((B, N, 1), jnp.float32),
    )(points, ptsT, frow, features)


def _knn_mean(Qb, P, qpts_ref, ptsT_ref, X_ref, mask_ref, key_scr, A_scr):
    q = qpts_ref[0]
    pT = ptsT_ref[0]
    qn = jnp.sum(q * q, axis=1, keepdims=True)
    pn = jnp.sum(pT * pT, axis=0, keepdims=True)
    cross = jnp.dot(q, pT, preferred_element_type=jnp.float32)
    mask = mask_ref[0]
    pn = pn + (1.0 - mask) * 1e9
    d2 = (qn + pn) - 2.0 * cross
    bits = jax.lax.bitcast_convert_type(d2, jnp.int32)
    key = bits ^ ((bits >> 31) & jnp.int32(0x7FFFFFFF))
    key_scr[...] = key
    lo0 = jnp.min(key, axis=1, keepdims=True) - 1
    hi0 = jnp.max(key, axis=1, keepdims=True)

    def _probe(mid, state):
        lo, hi, found, tg, cntlo, cnthi = state
        cnt = jnp.sum((key_scr[...] <= mid).astype(jnp.int32),
                      axis=1, keepdims=True)
        ge = cnt >= _K
        exact = (cnt == _K) & (found == 0)
        tg = jnp.where(exact, mid, tg)
        found = found | exact.astype(jnp.int32)
        return (jnp.where(ge, lo, mid), jnp.where(ge, mid, hi),
                found, tg,
                jnp.where(ge, cntlo, cnt), jnp.where(ge, cnt, cnthi))

    def bcond(state):
        lo, hi, found, _tg, _cl, _ch = state
        return jnp.any((found == 0) & ((hi - lo) != 1))

    def bstep(state):
        lo, hi, _f, _tg, _cl, _ch = state
        midb = (lo >> 1) + (hi >> 1) + (lo & hi & 1)
        state = _probe(midb, state)
        lo, hi, found, tg, cntlo, cnthi = state
        frac = ((_K - cntlo).astype(jnp.float32)
                / jnp.maximum((cnthi - cntlo).astype(jnp.float32), 1.0))
        mid_f = lo.astype(jnp.float32) + (hi - lo).astype(jnp.float32) * frac
        midi = jnp.clip(mid_f.astype(jnp.int32), lo + 1,
                        jnp.maximum(hi - 1, lo + 1))
        return _probe(midi, state)

    state0 = (lo0, hi0, jnp.zeros_like(lo0), hi0,
              jnp.zeros_like(lo0), jnp.full_like(lo0, P))
    _, hi_f, found, tg, _, cnthi = jax.lax.while_loop(bcond, bstep, state0)
    fnd = found != 0
    t = jnp.where(fnd, tg, hi_f)
    cnt_le = jnp.where(fnd, _K, cnthi)
    keyv = key_scr[...]
    le = keyv <= t
    A_scr[...] = le.astype(jnp.float32)

    @pl.when(jnp.max(cnt_le) > _K)
    def _tie_fill():
        kv = key_scr[...]
        need = _K - jnp.sum((kv < t).astype(jnp.int32), axis=1, keepdims=True)
        iota = jax.lax.broadcasted_iota(jnp.int32, (Qb, P), 1)

        def fstep(i, lohi):
            lo2, hi2 = lohi
            mid = (lo2 + hi2) >> 1
            c = jnp.sum(((key_scr[...] == t) & (iota < mid)).astype(jnp.int32),
                        axis=1, keepdims=True)
            geq = c >= need
            return jnp.where(geq, lo2, mid), jnp.where(geq, mid, hi2)

        z = jnp.zeros_like(need)
        _, m = jax.lax.fori_loop(0, 13, fstep, (z, z + P))
        kv2 = key_scr[...]
        A_scr[...] = ((kv2 < t) | ((kv2 == t) & (iota < m))).astype(jnp.float32)

    A = A_scr[...]
    mall = jnp.dot(A, X_ref[0], preferred_element_type=jnp.float32) * (1.0 / _K)
    return q, mall


def _layer_body(out_relu, Dx, Qb, P,
                qpts_ref, ptsT_ref, X_ref, mask_ref,
                Wall_ref, Wp_ref, b_ref, out_ref, inf_ref, key_scr, A_scr):
    q, mall = _knn_mean(Qb, P, qpts_ref, ptsT_ref, X_ref, mask_ref,
                        key_scr, A_scr)
    out = (jnp.dot(mall, Wall_ref[...], preferred_element_type=jnp.float32)
           - jnp.dot(q, Wp_ref[...], preferred_element_type=jnp.float32)
           + b_ref[...])
    if out_relu:
        out = jnp.maximum(out, 0.0)
    out_ref[0] = out
    inf_ref[0] = mall[:, Dx - 1:Dx]


def _tail_body(Dx, Qb, P,
               qpts_ref, ptsT_ref, X_ref, mask_ref,
               Wall_ref, Wp_ref, b_ref, qmask_ref,
               M0_ref, mb0_ref, M1_ref, mb1_ref, M2_ref, mb2_ref,
               M3_ref, mb3_ref, out_ref, key_scr, A_scr):
    q, mall = _knn_mean(Qb, P, qpts_ref, ptsT_ref, X_ref, mask_ref,
                        key_scr, A_scr)
    f = (jnp.dot(mall, Wall_ref[...], preferred_element_type=jnp.float32)
         - jnp.dot(q, Wp_ref[...], preferred_element_type=jnp.float32)
         + b_ref[...])
    in_f = mall[:, Dx - 1:Dx]
    valid = jnp.logical_and(in_f > 0.0, qmask_ref[0] > 0.0)
    vf = valid.astype(jnp.float32)
    h = vf * f
    h = jnp.maximum(jnp.dot(h, M0_ref[...], preferred_element_type=jnp.float32) + mb0_ref[...], 0.0)
    h = jnp.maximum(jnp.dot(h, M1_ref[...], preferred_element_type=jnp.float32) + mb1_ref[...], 0.0)
    h = jnp.maximum(jnp.dot(h, M2_ref[...], preferred_element_type=jnp.float32) + mb2_ref[...], 0.0)
    h = jnp.dot(h, M3_ref[...], preferred_element_type=jnp.float32) + mb3_ref[...]
    cnt = jnp.sum(vf, axis=0, keepdims=True)
    cnt = jnp.where(cnt == 0.0, 1.0, cnt)
    pooled = jnp.sum(h, axis=0, keepdims=True) / cnt
    out_ref[0] = 1.0 / (1.0 + jnp.exp(-pooled))


def _conv(l, pts, feats, smpl, maskcol, W, b):
    B, P, D = feats.shape
    Q = P // 2
    Qb = min(Q, 256)
    O = W.shape[1]
    Dx = D + 4
    q_pts = pts[:, ::2]
    ptsT = jnp.swapaxes(pts, 1, 2)
    maskrow = jnp.swapaxes(maskcol, 1, 2)
    X = jnp.concatenate([pts, feats, smpl], axis=-1)
    Wall = jnp.concatenate([W, jnp.zeros((1, O), W.dtype)], axis=0)
    Wp = W[:3]
    b2 = b[None, :]
    out, in_f = pl.pallas_call(
        functools.partial(_layer_body, l < 3, Dx, Qb, P),
        grid=(B, Q // Qb),
        in_specs=[
            pl.BlockSpec((1, Qb, 3), lambda b_, i: (b_, i, 0)),
            pl.BlockSpec((1, 3, P), lambda b_, i: (b_, 0, 0)),
            pl.BlockSpec((1, P, Dx), lambda b_, i: (b_, 0, 0)),
            pl.BlockSpec((1, 1, P), lambda b_, i: (b_, 0, 0)),
            pl.BlockSpec((Dx, O), lambda b_, i: (0, 0)),
            pl.BlockSpec((3, O), lambda b_, i: (0, 0)),
            pl.BlockSpec((1, O), lambda b_, i: (0, 0)),
        ],
        out_specs=[
            pl.BlockSpec((1, Qb, O), lambda b_, i: (b_, i, 0)),
            pl.BlockSpec((1, Qb, 1), lambda b_, i: (b_, i, 0)),
        ],
        out_shape=[
            jax.ShapeDtypeStruct((B, Q, O), jnp.float32),
            jax.ShapeDtypeStruct((B, Q, 1), jnp.float32),
        ],
        scratch_shapes=[
            pltpu.VMEM((Qb, P), jnp.int32),
            pltpu.VMEM((Qb, P), jnp.float32),
        ],
    )(q_pts, ptsT, X, maskrow, Wall, Wp, b2)
    return q_pts, out, maskcol[:, ::2], in_f


def _tail(pts, feats, smpl, maskcol, W, b, Ms, mbs):
    B, P, D = feats.shape
    Q = P // 2
    Qb = Q
    O = W.shape[1]
    Dx = D + 4
    q_pts = pts[:, ::2]
    ptsT = jnp.swapaxes(pts, 1, 2)
    maskrow = jnp.swapaxes(maskcol, 1, 2)
    X = jnp.concatenate([pts, feats, smpl], axis=-1)
    Wall = jnp.concatenate([W, jnp.zeros((1, O), W.dtype)], axis=0)
    Wp = W[:3]
    b2 = b[None, :]
    qmask = maskcol[:, ::2]
    specs = [
        pl.BlockSpec((1, Qb, 3), lambda b_: (b_, 0, 0)),
        pl.BlockSpec((1, 3, P), lambda b_: (b_, 0, 0)),
        pl.BlockSpec((1, P, Dx), lambda b_: (b_, 0, 0)),
        pl.BlockSpec((1, 1, P), lambda b_: (b_, 0, 0)),
        pl.BlockSpec((Dx, O), lambda b_: (0, 0)),
        pl.BlockSpec((3, O), lambda b_: (0, 0)),
        pl.BlockSpec((1, O), lambda b_: (0, 0)),
        pl.BlockSpec((1, Qb, 1), lambda b_: (b_, 0, 0)),
    ]
    args = [q_pts, ptsT, X, maskrow, Wall, Wp, b2, qmask]
    for M, mb in zip(Ms, mbs):
        specs.append(pl.BlockSpec(M.shape, lambda b_: (0, 0)))
        specs.append(pl.BlockSpec((1, mb.shape[0]), lambda b_: (0, 0)))
        args.append(M)
        args.append(mb[None, :])
    out = pl.pallas_call(
        functools.partial(_tail_body, Dx, Qb, P),
        grid=(B,),
        in_specs=specs,
        out_specs=pl.BlockSpec((1, 1, 1), lambda b_: (b_, 0, 0)),
        out_shape=jax.ShapeDtypeStruct((B, 1, 1), jnp.float32),
        scratch_shapes=[
            pltpu.VMEM((Qb, P), jnp.int32),
            pltpu.VMEM((Qb, P), jnp.float32),
        ],
    )(*args)
    return out.reshape(B)


def kernel(points, features, W0, b0, W1, b1, W2, b2, W3, b3,
           M0, mb0, M1, mb1, M2, mb2, M3, mb3):
    maskcol = _expand_mask(points, features)
    pts, feats, smpl = points, points, maskcol
    Ws = [W0, W1, W2]
    bs = [b0, b1, b2]
    for l in range(3):
        pts, feats, maskcol, smpl = _conv(l, pts, feats, smpl, maskcol, Ws[l], bs[l])
    return _tail(pts, feats, smpl, maskcol, W3, b3,
                 [M0, M1, M2, M3], [mb0, mb1, mb2, mb3])


# speedup vs baseline: 1.1422x; 1.1422x over previous
import functools

import jax
import jax.numpy as jnp
from jax.experimental import pallas as pl
from jax.experimental.pallas import tpu as pltpu

_K = 32
_EXPAND2 = 0.01


def _expand_body(qpts_ref, ptsT_ref, frow_ref, fcol_ref, out_ref):
    q = qpts_ref[0]
    pT = ptsT_ref[0]
    frow = frow_ref[0]
    m0row = frow > 0.0
    filtT = jnp.where(m0row, pT, 1e6)
    qn = jnp.sum(q * q, axis=1, keepdims=True)
    fn = jnp.sum(filtT * filtT, axis=0, keepdims=True)
    cross = jnp.dot(q, filtT, preferred_element_type=jnp.float32)
    d2 = qn + fn - 2.0 * cross
    nn = jnp.min(d2, axis=1, keepdims=True)
    m0q = fcol_ref[0] > 0.0
    emask = jnp.logical_or(m0q, nn < _EXPAND2)
    out_ref[0] = emask.astype(jnp.float32)


def _expand_mask(points, features):
    B, N, _ = points.shape
    Nb = 256
    ptsT = jnp.swapaxes(points, 1, 2)
    frow = jnp.swapaxes(features, 1, 2)
    return pl.pallas_call(
        _expand_body,
        grid=(B, N // Nb),
        in_specs=[
            pl.BlockSpec((1, Nb, 3), lambda b, i: (b, i, 0)),
            pl.BlockSpec((1, 3, N), lambda b, i: (b, 0, 0)),
            pl.BlockSpec((1, 1, N), lambda b, i: (b, 0, 0)),
            pl.BlockSpec((1, Nb, 1), lambda b, i: (b, i, 0)),
        ],
        out_specs=pl.BlockSpec((1, Nb, 1), lambda b, i: (b, i, 0)),
        out_shape=jax.ShapeDtypeStruct((B, N, 1), jnp.float32),
    )(points, ptsT, frow, features)


def _knn_mean(Qb, P, qpts_ref, ptsT_ref, X_ref, mask_ref, key_scr, A_scr):
    q = qpts_ref[0]
    pT = ptsT_ref[0]
    qn = jnp.sum(q * q, axis=1, keepdims=True)
    pn = jnp.sum(pT * pT, axis=0, keepdims=True)
    cross = jnp.dot(q, pT, preferred_element_type=jnp.float32)
    mask = mask_ref[0]
    pn = pn + (1.0 - mask) * 1e9
    d2 = (qn + pn) - 2.0 * cross
    bits = jax.lax.bitcast_convert_type(d2, jnp.int32)
    key = bits ^ ((bits >> 31) & jnp.int32(0x7FFFFFFF))
    key_scr[...] = key
    lo0 = jnp.min(key, axis=1, keepdims=True) - 1
    hi0 = jnp.max(key, axis=1, keepdims=True)

    def bcond(state):
        lo, hi, found, _tg, _ch = state
        return jnp.any((found == 0) & ((hi - lo) != 1))

    def bstep(state):
        lo, hi, found, tg, cnthi = state
        mid = (lo >> 1) + (hi >> 1) + (lo & hi & 1)
        cnt = jnp.sum((key_scr[...] <= mid).astype(jnp.int32),
                      axis=1, keepdims=True)
        ge = cnt >= _K
        exact = (cnt == _K) & (found == 0)
        tg = jnp.where(exact, mid, tg)
        found = found | exact.astype(jnp.int32)
        return (jnp.where(ge, lo, mid), jnp.where(ge, mid, hi),
                found, tg, jnp.where(ge, cnt, cnthi))

    state0 = (lo0, hi0, jnp.zeros_like(lo0), hi0, jnp.full_like(lo0, P))
    _, hi_f, found, tg, cnthi = jax.lax.while_loop(bcond, bstep, state0)
    fnd = found != 0
    t = jnp.where(fnd, tg, hi_f)
    cnt_le = jnp.where(fnd, _K, cnthi)
    keyv = key_scr[...]
    le = keyv <= t
    A_scr[...] = le.astype(jnp.float32)

    @pl.when(jnp.max(cnt_le) > _K)
    def _tie_fill():
        kv = key_scr[...]
        need = _K - jnp.sum((kv < t).astype(jnp.int32), axis=1, keepdims=True)
        iota = jax.lax.broadcasted_iota(jnp.int32, (Qb, P), 1)

        def fstep(i, lohi):
            lo2, hi2 = lohi
            mid = (lo2 + hi2) >> 1
            c = jnp.sum(((key_scr[...] == t) & (iota < mid)).astype(jnp.int32),
                        axis=1, keepdims=True)
            geq = c >= need
            return jnp.where(geq, lo2, mid), jnp.where(geq, mid, hi2)

        z = jnp.zeros_like(need)
        _, m = jax.lax.fori_loop(0, 13, fstep, (z, z + P))
        kv2 = key_scr[...]
        A_scr[...] = ((kv2 < t) | ((kv2 == t) & (iota < m))).astype(jnp.float32)

    A = A_scr[...]
    mall = jnp.dot(A, X_ref[0], preferred_element_type=jnp.float32) * (1.0 / _K)
    return q, mall


def _layer_body(out_relu, Dx, Qb, P,
                qpts_ref, ptsT_ref, X_ref, mask_ref,
                Wall_ref, Wp_ref, b_ref, xout_ref, key_scr, A_scr):
    q, mall = _knn_mean(Qb, P, qpts_ref, ptsT_ref, X_ref, mask_ref,
                        key_scr, A_scr)
    out = (jnp.dot(mall, Wall_ref[...], preferred_element_type=jnp.float32)
           - jnp.dot(q, Wp_ref[...], preferred_element_type=jnp.float32)
           + b_ref[...])
    if out_relu:
        out = jnp.maximum(out, 0.0)
    xout_ref[0] = jnp.concatenate([q, out, mall[:, Dx - 1:Dx]], axis=-1)


def _tail_body(Dx, Qb, P,
               qpts_ref, ptsT_ref, X_ref, mask_ref,
               Wall_ref, Wp_ref, b_ref, qmask_ref,
               M0_ref, mb0_ref, M1_ref, mb1_ref, M2_ref, mb2_ref,
               M3_ref, mb3_ref, out_ref, key_scr, A_scr):
    q, mall = _knn_mean(Qb, P, qpts_ref, ptsT_ref, X_ref, mask_ref,
                        key_scr, A_scr)
    f = (jnp.dot(mall, Wall_ref[...], preferred_element_type=jnp.float32)
         - jnp.dot(q, Wp_ref[...], preferred_element_type=jnp.float32)
         + b_ref[...])
    in_f = mall[:, Dx - 1:Dx]
    valid = jnp.logical_and(in_f > 0.0, qmask_ref[0] > 0.0)
    vf = valid.astype(jnp.float32)
    h = vf * f
    h = jnp.maximum(jnp.dot(h, M0_ref[...], preferred_element_type=jnp.float32) + mb0_ref[...], 0.0)
    h = jnp.maximum(jnp.dot(h, M1_ref[...], preferred_element_type=jnp.float32) + mb1_ref[...], 0.0)
    h = jnp.maximum(jnp.dot(h, M2_ref[...], preferred_element_type=jnp.float32) + mb2_ref[...], 0.0)
    h = jnp.dot(h, M3_ref[...], preferred_element_type=jnp.float32) + mb3_ref[...]
    cnt = jnp.sum(vf, axis=0, keepdims=True)
    cnt = jnp.where(cnt == 0.0, 1.0, cnt)
    pooled = jnp.sum(h, axis=0, keepdims=True) / cnt
    out_ref[0] = 1.0 / (1.0 + jnp.exp(-pooled))


def _conv(l, X, pts, maskcol, W, b):
    B, P, Dx = X.shape
    Q = P // 2
    Qb = min(Q, 256)
    O = W.shape[1]
    q_pts = pts[:, ::2]
    ptsT = jnp.swapaxes(pts, 1, 2)
    maskrow = jnp.swapaxes(maskcol, 1, 2)
    Wall = jnp.concatenate([W, jnp.zeros((1, O), W.dtype)], axis=0)
    Wp = W[:3]
    b2 = b[None, :]
    X_next = pl.pallas_call(
        functools.partial(_layer_body, True, Dx, Qb, P),
        grid=(B, Q // Qb),
        in_specs=[
            pl.BlockSpec((1, Qb, 3), lambda b_, i: (b_, i, 0)),
            pl.BlockSpec((1, 3, P), lambda b_, i: (b_, 0, 0)),
            pl.BlockSpec((1, P, Dx), lambda b_, i: (b_, 0, 0)),
            pl.BlockSpec((1, 1, P), lambda b_, i: (b_, 0, 0)),
            pl.BlockSpec((Dx, O), lambda b_, i: (0, 0)),
            pl.BlockSpec((3, O), lambda b_, i: (0, 0)),
            pl.BlockSpec((1, O), lambda b_, i: (0, 0)),
        ],
        out_specs=pl.BlockSpec((1, Qb, O + 4), lambda b_, i: (b_, i, 0)),
        out_shape=jax.ShapeDtypeStruct((B, Q, O + 4), jnp.float32),
        scratch_shapes=[
            pltpu.VMEM((Qb, P), jnp.int32),
            pltpu.VMEM((Qb, P), jnp.float32),
        ],
    )(q_pts, ptsT, X, maskrow, Wall, Wp, b2)
    return X_next


def _tail(X, pts, maskcol, W, b, Ms, mbs):
    B, P, Dx = X.shape
    Q = P // 2
    Qb = Q
    O = W.shape[1]
    q_pts = pts[:, ::2]
    ptsT = jnp.swapaxes(pts, 1, 2)
    maskrow = jnp.swapaxes(maskcol, 1, 2)
    Wall = jnp.concatenate([W, jnp.zeros((1, O), W.dtype)], axis=0)
    Wp = W[:3]
    b2 = b[None, :]
    qmask = maskcol[:, ::2]
    specs = [
        pl.BlockSpec((1, Qb, 3), lambda b_: (b_, 0, 0)),
        pl.BlockSpec((1, 3, P), lambda b_: (b_, 0, 0)),
        pl.BlockSpec((1, P, Dx), lambda b_: (b_, 0, 0)),
        pl.BlockSpec((1, 1, P), lambda b_: (b_, 0, 0)),
        pl.BlockSpec((Dx, O), lambda b_: (0, 0)),
        pl.BlockSpec((3, O), lambda b_: (0, 0)),
        pl.BlockSpec((1, O), lambda b_: (0, 0)),
        pl.BlockSpec((1, Qb, 1), lambda b_: (b_, 0, 0)),
    ]
    args = [q_pts, ptsT, X, maskrow, Wall, Wp, b2, qmask]
    for M, mb in zip(Ms, mbs):
        specs.append(pl.BlockSpec(M.shape, lambda b_: (0, 0)))
        specs.append(pl.BlockSpec((1, mb.shape[0]), lambda b_: (0, 0)))
        args.append(M)
        args.append(mb[None, :])
    out = pl.pallas_call(
        functools.partial(_tail_body, Dx, Qb, P),
        grid=(B,),
        in_specs=specs,
        out_specs=pl.BlockSpec((1, 1, 1), lambda b_: (b_, 0, 0)),
        out_shape=jax.ShapeDtypeStruct((B, 1, 1), jnp.float32),
        scratch_shapes=[
            pltpu.VMEM((Qb, P), jnp.int32),
            pltpu.VMEM((Qb, P), jnp.float32),
        ],
    )(*args)
    return out.reshape(B)


def kernel(points, features, W0, b0, W1, b1, W2, b2, W3, b3,
           M0, mb0, M1, mb1, M2, mb2, M3, mb3):
    maskcol = _expand_mask(points, features)
    X = jnp.concatenate([points, points, maskcol], axis=-1)
    pts = points
    Ws = [W0, W1, W2]
    bs = [b0, b1, b2]
    for l in range(3):
        X = _conv(l, X, pts, maskcol, Ws[l], bs[l])
        pts = pts[:, ::2]
        maskcol = maskcol[:, ::2]
    return _tail(X, pts, maskcol, W3, b3,
                 [M0, M1, M2, M3], [mb0, mb1, mb2, mb3])
